# baseline (speedup 1.0000x reference)
import functools

import jax
import jax.numpy as jnp
from jax import lax
from jax.experimental import pallas as pl
from jax.experimental.pallas import tpu as pltpu
from jax.experimental.pallas import tpu_sc as plsc

EMBED_DIM = 64
NUM_TOKENS = 16384
SEQ = 50
TABLE_ROWS = 1000000

TC_CH = 8192
CHUNK = 256
TBL = CHUNK // 128


def _tc_detile(w_t):
    nblk = (TABLE_ROWS + TC_CH - 1) // TC_CH

    def body(wt_ref, out_ref):
        y = wt_ref[...].T.reshape(TC_CH // 2, 2, EMBED_DIM)
        out_ref[:, 0:EMBED_DIM] = y[:, 0, :]
        out_ref[:, EMBED_DIM : 2 * EMBED_DIM] = y[:, 1, :]

    return pl.pallas_call(
        body,
        grid=(nblk,),
        in_specs=[pl.BlockSpec((EMBED_DIM, TC_CH), lambda i: (0, i))],
        out_specs=pl.BlockSpec((TC_CH // 2, 128), lambda i: (i, 0)),
        out_shape=jax.ShapeDtypeStruct((500032, 128), jnp.float32),
    )(w_t)


def _sc_gather(table, ids_t):
    info = plsc.get_sparse_core_info()
    nw = info.num_cores * info.num_subcores
    b_total = NUM_TOKENS * SEQ
    b_per_w = b_total // nw
    n_chunks = b_per_w // CHUNK
    n_pairs = n_chunks // 2
    mesh = plsc.VectorSubcoreMesh(core_axis_name="c", subcore_axis_name="s")

    pchunk = TBL * 1024
    psz = 8 * pchunk

    @functools.partial(
        pl.kernel,
        mesh=mesh,
        out_type=jax.ShapeDtypeStruct((SEQ * 8 * 128 * 8 * 128,), jnp.float32),
        compiler_params=pltpu.CompilerParams(
            use_tc_tiling_on_sc=False, needs_layout_passes=False
        ),
        scratch_types=[
            pltpu.VMEM((b_per_w,), jnp.int32),
            pltpu.VMEM((CHUNK, EMBED_DIM), jnp.float32),
            pltpu.VMEM((CHUNK, EMBED_DIM), jnp.float32),
            pltpu.VMEM((psz,), jnp.float32),
            pltpu.VMEM((psz,), jnp.float32),
            pltpu.SemaphoreType.DMA,
            pltpu.SemaphoreType.DMA,
            pltpu.SemaphoreType.DMA,
            pltpu.SemaphoreType.DMA,
        ],
    )
    def k(table_hbm, ids_hbm, out_hbm, idx_v, r0, r1, p0, p1, sg0, sg1, sp0, sp1):
        rows = (r0, r1)
        planes = (p0, p1)
        sg = (sg0, sg1)
        sp = (sp0, sp1)
        wid = lax.axis_index("s") * info.num_cores + lax.axis_index("c")
        w_base = wid * b_per_w
        pltpu.sync_copy(ids_hbm.at[pl.ds(w_base, b_per_w)], idx_v)
        iota = lax.iota(jnp.int32, 16)
        pidx = []
        for q in range(4):
            cvec = iota + 16 * q
            pidx.append((cvec >> 3) * pchunk + (cvec & 7) * 128)

        def gather_start(i, b):
            pltpu.async_copy(
                table_hbm.at[idx_v.at[pl.ds(i * CHUNK, CHUNK)]], rows[b], sg[b]
            )

        def gather_wait(b):
            pltpu.make_async_copy(
                table_hbm.at[idx_v.at[pl.ds(0, CHUNK)]], rows[b], sg[b]
            ).wait()

        def planes_start(i, b):
            f0 = w_base + i * CHUNK
            s = f0 // NUM_TOKENS
            tb0 = (f0 % NUM_TOKENS) // 128
            for c8 in range(8):
                pltpu.async_copy(
                    planes[b].at[pl.ds(c8 * pchunk, pchunk)],
                    out_hbm.at[pl.ds(((s * 8 + c8) * 128 + tb0) * 1024, pchunk)],
                    sp[b],
                )

        def planes_wait(b):
            pltpu.make_async_copy(out_hbm.at[pl.ds(0, psz)], planes[b], sp[b]).wait()

        def transpose(b):
            rv = rows[b]
            pv = planes[b]

            def j_body(j, carry):
                off = ((j >> 7) << 10) + (j & 127)
                for q in range(4):
                    vec = rv[j, pl.ds(q * 16, 16)]
                    plsc.store_scatter(pv, [pidx[q] + off], vec)
                return carry

            lax.fori_loop(0, CHUNK, j_body, 0, unroll=8)

        for b in range(2):
            gather_start(b, b)

        for b in range(2):
            gather_wait(b)
            transpose(b)
            planes_start(b, b)
            gather_start(b + 2, b)

        def pair(g, carry):
            for b in range(2):
                i = 2 * g + b
                gather_wait(b)
                planes_wait(b)
                transpose(b)
                planes_start(i, b)
                gather_start(i + 2, b)
            return carry

        lax.fori_loop(1, n_pairs - 1, pair, 0, unroll=False)

        for b in range(2):
            i = n_chunks - 2 + b
            gather_wait(b)
            planes_wait(b)
            transpose(b)
            planes_start(i, b)
        for b in range(2):
            planes_wait(b)

    return k(table, ids_t)


def kernel(token_ids, weight):
    w_t = weight.T
    lin = _tc_detile(w_t)
    table = lin.reshape(1000064, EMBED_DIM)
    ids_t = token_ids.T.reshape(NUM_TOKENS * SEQ).astype(jnp.int32)
    out_flat = _sc_gather(table, ids_t)
    out5 = out_flat.reshape(SEQ, 8, 128, 8, 128)
    return jnp.transpose(out5, (2, 4, 0, 1, 3)).reshape(NUM_TOKENS, SEQ, EMBED_DIM)


# speedup vs baseline: 1.0908x; 1.0908x over previous
import functools

import jax
import jax.numpy as jnp
from jax import lax
from jax.experimental import pallas as pl
from jax.experimental.pallas import tpu as pltpu
from jax.experimental.pallas import tpu_sc as plsc

EMBED_DIM = 64
NUM_TOKENS = 16384
SEQ = 50
TABLE_ROWS = 1000000

TC_CH = 8192
CHUNK = 256
TBL = CHUNK // 128


def _tc_detile(w_t):
    nblk = (TABLE_ROWS + TC_CH - 1) // TC_CH

    def body(wt_ref, out_ref):
        y = wt_ref[...].T.reshape(TC_CH // 2, 2, EMBED_DIM)
        out_ref[:, 0:EMBED_DIM] = y[:, 0, :]
        out_ref[:, EMBED_DIM : 2 * EMBED_DIM] = y[:, 1, :]

    return pl.pallas_call(
        body,
        grid=(nblk,),
        in_specs=[pl.BlockSpec((EMBED_DIM, TC_CH), lambda i: (0, i))],
        out_specs=pl.BlockSpec((TC_CH // 2, 128), lambda i: (i, 0)),
        out_shape=jax.ShapeDtypeStruct((500032, 128), jnp.float32),
    )(w_t)


def _sc_gather(table, ids_t):
    info = plsc.get_sparse_core_info()
    nw = info.num_cores * info.num_subcores
    b_total = NUM_TOKENS * SEQ
    b_per_w = b_total // nw
    n_chunks = b_per_w // CHUNK
    n_pairs = n_chunks // 2
    mesh = plsc.VectorSubcoreMesh(core_axis_name="c", subcore_axis_name="s")

    pchunk = TBL * 1024
    psz = 8 * pchunk

    @functools.partial(
        pl.kernel,
        mesh=mesh,
        out_type=jax.ShapeDtypeStruct((SEQ * 8 * 128 * 8 * 128,), jnp.float32),
        compiler_params=pltpu.CompilerParams(
            use_tc_tiling_on_sc=False, needs_layout_passes=False
        ),
        scratch_types=[
            pltpu.VMEM((b_per_w,), jnp.int32),
            pltpu.VMEM((CHUNK, EMBED_DIM), jnp.float32),
            pltpu.VMEM((CHUNK, EMBED_DIM), jnp.float32),
            pltpu.VMEM((psz,), jnp.float32),
            pltpu.VMEM((psz,), jnp.float32),
            pltpu.SemaphoreType.DMA,
            pltpu.SemaphoreType.DMA,
            pltpu.SemaphoreType.DMA,
            pltpu.SemaphoreType.DMA,
        ],
    )
    def k(table_hbm, ids_hbm, out_hbm, idx_v, r0, r1, p0, p1, sg0, sg1, sp0, sp1):
        rows = (r0, r1)
        planes = (p0, p1)
        sg = (sg0, sg1)
        sp = (sp0, sp1)
        wid = lax.axis_index("s") * info.num_cores + lax.axis_index("c")
        w_base = wid * b_per_w
        pltpu.sync_copy(ids_hbm.at[pl.ds(w_base, b_per_w)], idx_v)
        iota = lax.iota(jnp.int32, 16)
        pidx = []
        for q in range(4):
            cvec = iota + 16 * q
            pidx.append((cvec >> 3) * pchunk + (cvec & 7) * 128)

        def gather_start(i, b):
            pltpu.async_copy(
                table_hbm.at[idx_v.at[pl.ds(i * CHUNK, CHUNK)]], rows[b], sg[b]
            )

        def gather_wait(b):
            pltpu.make_async_copy(
                table_hbm.at[idx_v.at[pl.ds(0, CHUNK)]], rows[b], sg[b]
            ).wait()

        def planes_start(i, b):
            f0 = w_base + i * CHUNK
            s = f0 // NUM_TOKENS
            tb0 = (f0 % NUM_TOKENS) // 128
            for c8 in range(8):
                pltpu.async_copy(
                    planes[b].at[pl.ds(c8 * pchunk, pchunk)],
                    out_hbm.at[pl.ds(((s * 8 + c8) * 128 + tb0) * 1024, pchunk)],
                    sp[b],
                )

        def planes_wait(b):
            pltpu.make_async_copy(out_hbm.at[pl.ds(0, psz)], planes[b], sp[b]).wait()

        def transpose(b):
            rv = rows[b]
            pv = planes[b]

            @plsc.parallel_loop(0, CHUNK, 1, unroll=8)
            def _(j):
                off = ((j >> 7) << 10) + (j & 127)
                vecs = [rv[j, pl.ds(q * 16, 16)] for q in range(4)]
                for q in range(4):
                    plsc.store_scatter(pv, [pidx[q] + off], vecs[q])

        for b in range(2):
            gather_start(b, b)

        for b in range(2):
            gather_wait(b)
            transpose(b)
            planes_start(b, b)
            gather_start(b + 2, b)

        def pair(g, carry):
            for b in range(2):
                i = 2 * g + b
                gather_wait(b)
                planes_wait(b)
                transpose(b)
                planes_start(i, b)
                gather_start(i + 2, b)
            return carry

        lax.fori_loop(1, n_pairs - 1, pair, 0, unroll=False)

        for b in range(2):
            i = n_chunks - 2 + b
            gather_wait(b)
            planes_wait(b)
            transpose(b)
            planes_start(i, b)
        for b in range(2):
            planes_wait(b)

    return k(table, ids_t)


def kernel(token_ids, weight):
    w_t = weight.T
    lin = _tc_detile(w_t)
    table = lin.reshape(1000064, EMBED_DIM)
    ids_t = token_ids.T.reshape(NUM_TOKENS * SEQ).astype(jnp.int32)
    out_flat = _sc_gather(table, ids_t)
    out5 = out_flat.reshape(SEQ, 8, 128, 8, 128)
    return jnp.transpose(out5, (2, 4, 0, 1, 3)).reshape(NUM_TOKENS, SEQ, EMBED_DIM)
